# baseline (speedup 1.0000x reference)
import functools

import jax
import jax.numpy as jnp
from jax import lax
from jax.experimental import pallas as pl
from jax.experimental.pallas import tpu as pltpu
from jax.experimental.pallas import tpu_sc as plsc

NUM_ACTIONS = 1000000
ENC_DIM = 64
BATCH = 16384

_PAIR_W = 2 * ENC_DIM
_N_PAIRS = NUM_ACTIONS // 2

_info = plsc.get_sparse_core_info()
_NC, _NS = _info.num_cores, _info.num_subcores
_NW = _NC * _NS
_B_PER_W = BATCH // _NW
_CHUNK = 128
_N_CHUNKS = _B_PER_W // _CHUNK


@functools.partial(
    pl.kernel,
    mesh=plsc.VectorSubcoreMesh(core_axis_name="c", subcore_axis_name="s"),
    out_type=jax.ShapeDtypeStruct((BATCH, _PAIR_W), jnp.float32),
    scratch_types=[
        pltpu.VMEM((_B_PER_W,), jnp.int32),
        pltpu.VMEM((_B_PER_W, _PAIR_W), jnp.float32),
        pltpu.SemaphoreType.DMA,
    ],
    compiler_params=pltpu.CompilerParams(use_tc_tiling_on_sc=True),
)
def _sc_gather(pairs_hbm, idx_hbm, out_hbm, idx_v, rows_v, sem):
    wid = lax.axis_index("s") * _NC + lax.axis_index("c")
    base = wid * _B_PER_W
    pltpu.sync_copy(idx_hbm.at[pl.ds(base, _B_PER_W)], idx_v)
    copies = []
    for j in range(_N_CHUNKS):
        copies.append(
            pltpu.async_copy(
                pairs_hbm.at[idx_v.at[pl.ds(j * _CHUNK, _CHUNK)]],
                rows_v.at[pl.ds(j * _CHUNK, _CHUNK)],
                sem,
            )
        )
    for c in copies:
        c.wait()
    pltpu.sync_copy(rows_v, out_hbm.at[pl.ds(base, _B_PER_W)])


def kernel(act, table):
    act = act.astype(jnp.int32)
    pairs = jnp.reshape(table, (_N_PAIRS, _PAIR_W))
    wide = _sc_gather(pairs, act >> 1)
    return jnp.where(
        (act & 1)[:, None] == 1, wide[:, ENC_DIM:], wide[:, :ENC_DIM]
    )


# speedup vs baseline: 2.5520x; 2.5520x over previous
import functools

import jax
import jax.numpy as jnp
from jax import lax
from jax.experimental import pallas as pl
from jax.experimental.pallas import tpu as pltpu
from jax.experimental.pallas import tpu_sc as plsc

NUM_ACTIONS = 1000000
ENC_DIM = 64
BATCH = 16384

_SUBL = 8
_N_TILES = NUM_ACTIONS // _SUBL

_info = plsc.get_sparse_core_info()
_NC, _NS = _info.num_cores, _info.num_subcores
_NW = _NC * _NS
_B_PER_W = BATCH // _NW


@functools.partial(
    pl.kernel,
    mesh=plsc.VectorSubcoreMesh(core_axis_name="c", subcore_axis_name="s"),
    out_type=jax.ShapeDtypeStruct((BATCH, ENC_DIM), jnp.float32),
    scratch_types=[
        pltpu.VMEM((_B_PER_W,), jnp.int32),
        pltpu.VMEM((_B_PER_W, ENC_DIM), jnp.float32),
        pltpu.SemaphoreType.DMA,
    ],
    compiler_params=pltpu.CompilerParams(use_tc_tiling_on_sc=True),
)
def _sc_gather(tiles_hbm, act_hbm, out_hbm, idx_v, rows_v, sem):
    wid = lax.axis_index("s") * _NC + lax.axis_index("c")
    base = wid * _B_PER_W
    pltpu.sync_copy(act_hbm.at[pl.ds(base, _B_PER_W)], idx_v)

    @pl.loop(0, _B_PER_W, unroll=4)
    def _issue(r):
        a = idx_v[pl.ds(r, 1)][0]
        pltpu.async_copy(
            tiles_hbm.at[a >> 3, a & 7],
            rows_v.at[r],
            sem,
        )

    @pl.loop(0, _B_PER_W, unroll=4)
    def _drain(r):
        pltpu.make_async_copy(tiles_hbm.at[0, 0], rows_v.at[0], sem).wait()

    pltpu.sync_copy(rows_v, out_hbm.at[pl.ds(base, _B_PER_W)])


def kernel(act, table):
    act = act.astype(jnp.int32)
    tiles = jnp.reshape(table, (_N_TILES, _SUBL, ENC_DIM))
    return _sc_gather(tiles, act)
